# baseline (speedup 1.0000x reference)
import functools

import jax
import jax.numpy as jnp
from jax import lax
from jax.experimental import pallas as pl
from jax.experimental.pallas import tpu as pltpu
from jax.experimental.pallas import tpu_sc as plsc

F32 = jnp.float32
BF16 = jnp.bfloat16
_PREC = lax.Precision.DEFAULT

_N_NODES = 10000
_N_EDGES = 320000
_EDGE_DIM = 16
_NODE_DIM = 128
_HID = 64
_N_GRAPHS = 16

_NC = 2
_NS = 16
_NW = _NC * _NS
_EPW = _N_EDGES // _NW
_CHUNK = 80
_NCHUNK = _EPW // _CHUNK

_BE = 6400
_NBLK = _N_EDGES // _BE


def _prep_body(nodes_ref, w1s_ref, w1d_ref, gg_ref, w1u_ref, b1_ref,
               ps_ref, pd_ref, g_ref):
    n = nodes_ref[...]
    ps_ref[...] = jnp.dot(n, w1s_ref[...], precision=_PREC,
                          preferred_element_type=F32).astype(BF16)
    pd_ref[...] = jnp.dot(n, w1d_ref[...], precision=_PREC,
                          preferred_element_type=F32).astype(BF16)
    g_ref[...] = jnp.dot(gg_ref[...], w1u_ref[...], precision=_PREC,
                         preferred_element_type=F32) + b1_ref[...]


def _gather_body(ps_hbm, pd_hbm, send_hbm, recv_hbm,
                 out_hbm, idxs, idxr, bufs, sems):
    wid = lax.axis_index("s") * _NC + lax.axis_index("c")
    base = wid * _EPW
    pltpu.sync_copy(send_hbm.at[pl.ds(base, _EPW)], idxs)
    pltpu.sync_copy(recv_hbm.at[pl.ds(base, _EPW)], idxr)

    def _gather_copies(b, c):
        r0, r1 = bufs[b]
        semg, _ = sems[b]
        sl = pl.ds(c * _CHUNK, _CHUNK)
        return (
            pltpu.make_async_copy(ps_hbm.at[idxs.at[sl]], r0, semg),
            pltpu.make_async_copy(pd_hbm.at[idxr.at[sl]], r1, semg),
        )

    def _write_copies(b, c):
        r0, r1 = bufs[b]
        _, semw = sems[b]
        off = base + c * _CHUNK
        return (
            pltpu.make_async_copy(
                r0, out_hbm.at[pl.ds(off, _CHUNK), pl.ds(0, _HID)], semw),
            pltpu.make_async_copy(
                r1, out_hbm.at[pl.ds(off, _CHUNK), pl.ds(_HID, _HID)], semw),
        )

    def fire(b, c):
        for cp in _gather_copies(b, c):
            cp.start()

    def waitg(b, c):
        for cp in _gather_copies(b, c):
            cp.wait()

    def firew(b, c):
        for cp in _write_copies(b, c):
            cp.start()

    def waitw(b, c):
        for cp in _write_copies(b, c):
            cp.wait()

    fire(0, 0)
    fire(1, 1)
    waitg(0, 0)
    firew(0, 0)
    waitg(1, 1)
    firew(1, 1)
    waitw(0, 0)
    fire(0, 2)

    @pl.loop(1, _NCHUNK // 2)
    def _pair(k):
        c0 = 2 * k
        waitw(1, c0 - 1)
        fire(1, c0 + 1)
        waitg(0, c0)
        firew(0, c0)
        waitg(1, c0 + 1)
        firew(1, c0 + 1)
        waitw(0, c0)
        fire(0, c0 + 2)

    last = _NCHUNK - 1
    waitg(0, last)
    firew(0, last)
    waitw(0, last)
    waitw(1, last - 1)


def _mlp_body(s2_ref, et_ref, b3_ref, g_ref, j_ref, w1e_ref, w2_ref, b2_ref,
              ot_ref):
    brow = jnp.broadcast_to(b3_ref[0], (_N_GRAPHS, _BE))
    onehot_t = (brow == lax.broadcasted_iota(jnp.int32, (_N_GRAPHS, _BE), 0)
                ).astype(F32)
    x = (jnp.dot(s2_ref[...], j_ref[...], precision=_PREC,
                 preferred_element_type=F32)
         + lax.dot_general(et_ref[...], w1e_ref[...],
                           (((0,), (0,)), ((), ())), precision=_PREC,
                           preferred_element_type=F32)
         + lax.dot_general(onehot_t, g_ref[...],
                           (((0,), (0,)), ((), ())), precision=_PREC,
                           preferred_element_type=F32))
    h = jnp.maximum(x, 0.0)
    ot_ref[...] = lax.dot_general(w2_ref[...], h,
                                  (((0,), (1,)), ((), ())), precision=_PREC,
                                  preferred_element_type=F32) + b2_ref[...]


def kernel(nodes, edges, graph_globals, W1, b1, W2, b2, edge_index,
           batch_edges):
    W1e = W1[:_EDGE_DIM]
    W1s = W1[_EDGE_DIM:_EDGE_DIM + _NODE_DIM]
    W1d = W1[_EDGE_DIM + _NODE_DIM:_EDGE_DIM + 2 * _NODE_DIM]
    W1u = W1[_EDGE_DIM + 2 * _NODE_DIM:]
    send = edge_index[0]
    recv = edge_index[1]
    b1r = b1.reshape(1, _HID)
    b2c = b2.reshape(_EDGE_DIM, 1)
    jmat = jnp.concatenate([jnp.eye(_HID, dtype=BF16),
                            jnp.eye(_HID, dtype=BF16)], axis=0)
    edges_t = edges.T
    batch3 = batch_edges.reshape(_NBLK, 1, _BE)

    ps, pd, g = pl.pallas_call(
        _prep_body,
        out_shape=[
            jax.ShapeDtypeStruct((_N_NODES, _HID), BF16),
            jax.ShapeDtypeStruct((_N_NODES, _HID), BF16),
            jax.ShapeDtypeStruct((_N_GRAPHS, _HID), F32),
        ],
    )(nodes, W1s, W1d, graph_globals, W1u, b1r)

    mesh = plsc.VectorSubcoreMesh(core_axis_name="c", subcore_axis_name="s",
                                  num_cores=_NC, num_subcores=_NS)

    def _gather_entry(ps_hbm, pd_hbm, send_hbm, recv_hbm,
                      out_hbm, idxs, idxr,
                      r0a, r1a, r0b, r1b, sga, swa, sgb, swb):
        _gather_body(ps_hbm, pd_hbm, send_hbm, recv_hbm,
                     out_hbm, idxs, idxr,
                     [(r0a, r1a), (r0b, r1b)],
                     [(sga, swa), (sgb, swb)])

    gather = functools.partial(
        pl.kernel,
        mesh=mesh,
        compiler_params=pltpu.CompilerParams(use_tc_tiling_on_sc=False),
        out_type=jax.ShapeDtypeStruct((_N_EDGES, 2 * _HID), BF16),
        scratch_types=[
            pltpu.VMEM((_EPW,), jnp.int32),
            pltpu.VMEM((_EPW,), jnp.int32),
            pltpu.VMEM((_CHUNK, _HID), BF16),
            pltpu.VMEM((_CHUNK, _HID), BF16),
            pltpu.VMEM((_CHUNK, _HID), BF16),
            pltpu.VMEM((_CHUNK, _HID), BF16),
            pltpu.SemaphoreType.DMA,
            pltpu.SemaphoreType.DMA,
            pltpu.SemaphoreType.DMA,
            pltpu.SemaphoreType.DMA,
        ],
    )(_gather_entry)
    s2 = gather(ps, pd, send, recv)

    out_t = pl.pallas_call(
        _mlp_body,
        grid=(_NBLK,),
        in_specs=[
            pl.BlockSpec((_BE, 2 * _HID), lambda i: (i, 0)),
            pl.BlockSpec((_EDGE_DIM, _BE), lambda i: (0, i)),
            pl.BlockSpec((1, 1, _BE), lambda i: (i, 0, 0)),
            pl.BlockSpec((_N_GRAPHS, _HID), lambda i: (0, 0)),
            pl.BlockSpec((2 * _HID, _HID), lambda i: (0, 0)),
            pl.BlockSpec((_EDGE_DIM, _HID), lambda i: (0, 0)),
            pl.BlockSpec((_HID, _EDGE_DIM), lambda i: (0, 0)),
            pl.BlockSpec((_EDGE_DIM, 1), lambda i: (0, 0)),
        ],
        out_specs=pl.BlockSpec((_EDGE_DIM, _BE), lambda i: (0, i)),
        out_shape=jax.ShapeDtypeStruct((_EDGE_DIM, _N_EDGES), F32),
    )(s2, edges_t, batch3, g, jmat, W1e, W2, b2c)
    return out_t.T


# speedup vs baseline: 2.0334x; 2.0334x over previous
import functools

import jax
import jax.numpy as jnp
from jax import lax
from jax.experimental import pallas as pl
from jax.experimental.pallas import tpu as pltpu
from jax.experimental.pallas import tpu_sc as plsc

F32 = jnp.float32
BF16 = jnp.bfloat16
_PREC = lax.Precision.DEFAULT

_N_NODES = 10000
_N_EDGES = 320000
_EDGE_DIM = 16
_NODE_DIM = 128
_HID = 64
_N_GRAPHS = 16

_NSLICE = 5
_ESL = _N_EDGES // _NSLICE

_NC = 2
_NS = 16
_NW = _NC * _NS
_EPW = _ESL // _NW
_CHUNK = 80
_NCHUNK = _EPW // _CHUNK

_BE = 6400
_NBLK = _ESL // _BE
_NBLK_ALL = _N_EDGES // _BE


def _prep_body(nodes_ref, w1s_ref, w1d_ref, gg_ref, w1u_ref, b1_ref,
               ps_ref, pd_ref, g_ref):
    n = nodes_ref[...]
    ps_ref[...] = jnp.dot(n, w1s_ref[...], precision=_PREC,
                          preferred_element_type=F32)
    pd_ref[...] = jnp.dot(n, w1d_ref[...], precision=_PREC,
                          preferred_element_type=F32)
    g_ref[...] = jnp.dot(gg_ref[...], w1u_ref[...], precision=_PREC,
                         preferred_element_type=F32) + b1_ref[...]


def _gather_body(ps_hbm, pd_hbm, send_hbm, recv_hbm,
                 out_hbm, idxs, idxr, bufs, sems):
    wid = lax.axis_index("s") * _NC + lax.axis_index("c")
    base = wid * _EPW
    pltpu.sync_copy(send_hbm.at[pl.ds(base, _EPW)], idxs)
    pltpu.sync_copy(recv_hbm.at[pl.ds(base, _EPW)], idxr)

    def _gather_copies(b, c):
        r0, r1 = bufs[b]
        semg, _ = sems[b]
        sl = pl.ds(c * _CHUNK, _CHUNK)
        return (
            pltpu.make_async_copy(ps_hbm.at[idxs.at[sl]], r0, semg),
            pltpu.make_async_copy(pd_hbm.at[idxr.at[sl]], r1, semg),
        )

    def _write_copies(b, c):
        r0, r1 = bufs[b]
        _, semw = sems[b]
        off = base + c * _CHUNK
        return (
            pltpu.make_async_copy(
                r0, out_hbm.at[pl.ds(off, _CHUNK), pl.ds(0, _HID)], semw),
            pltpu.make_async_copy(
                r1, out_hbm.at[pl.ds(off, _CHUNK), pl.ds(_HID, _HID)], semw),
        )

    def fire(b, c):
        for cp in _gather_copies(b, c):
            cp.start()

    def waitg(b, c):
        for cp in _gather_copies(b, c):
            cp.wait()

    def firew(b, c):
        for cp in _write_copies(b, c):
            cp.start()

    def waitw(b, c):
        for cp in _write_copies(b, c):
            cp.wait()

    fire(0, 0)
    fire(1, 1)
    waitg(0, 0)
    firew(0, 0)
    waitg(1, 1)
    firew(1, 1)
    waitw(0, 0)
    fire(0, 2)

    @pl.loop(1, _NCHUNK // 2)
    def _pair(k):
        c0 = 2 * k
        waitw(1, c0 - 1)
        fire(1, c0 + 1)
        waitg(0, c0)
        firew(0, c0)
        waitg(1, c0 + 1)
        firew(1, c0 + 1)
        waitw(0, c0)
        fire(0, c0 + 2)

    last = _NCHUNK - 1
    waitg(0, last)
    firew(0, last)
    waitw(0, last)
    waitw(1, last - 1)


def _mlp_body(s2_ref, et_ref, b3_ref, g_ref, j_ref, w1e_ref, w2_ref, b2_ref,
              ot_ref):
    brow = jnp.broadcast_to(b3_ref[0], (_N_GRAPHS, _BE))
    onehot_t = (brow == lax.broadcasted_iota(jnp.int32, (_N_GRAPHS, _BE), 0)
                ).astype(F32)
    x = (jnp.dot(s2_ref[...], j_ref[...], precision=_PREC,
                 preferred_element_type=F32)
         + lax.dot_general(et_ref[...], w1e_ref[...],
                           (((0,), (0,)), ((), ())), precision=_PREC,
                           preferred_element_type=F32)
         + lax.dot_general(onehot_t, g_ref[...],
                           (((0,), (0,)), ((), ())), precision=_PREC,
                           preferred_element_type=F32))
    h = jnp.maximum(x, 0.0)
    ot_ref[...] = lax.dot_general(w2_ref[...], h,
                                  (((0,), (1,)), ((), ())), precision=_PREC,
                                  preferred_element_type=F32) + b2_ref[...]


def kernel(nodes, edges, graph_globals, W1, b1, W2, b2, edge_index,
           batch_edges):
    W1e = W1[:_EDGE_DIM]
    W1s = W1[_EDGE_DIM:_EDGE_DIM + _NODE_DIM]
    W1d = W1[_EDGE_DIM + _NODE_DIM:_EDGE_DIM + 2 * _NODE_DIM]
    W1u = W1[_EDGE_DIM + 2 * _NODE_DIM:]
    send = edge_index[0]
    recv = edge_index[1]
    b1r = b1.reshape(1, _HID)
    b2c = b2.reshape(_EDGE_DIM, 1)
    jmat = jnp.concatenate([jnp.eye(_HID, dtype=F32),
                            jnp.eye(_HID, dtype=F32)], axis=0)
    edges_t = edges.T
    batch3 = batch_edges.reshape(_NBLK_ALL, 1, _BE)

    ps, pd, g = pl.pallas_call(
        _prep_body,
        out_shape=[
            jax.ShapeDtypeStruct((_N_NODES, _HID), F32),
            jax.ShapeDtypeStruct((_N_NODES, _HID), F32),
            jax.ShapeDtypeStruct((_N_GRAPHS, _HID), F32),
        ],
    )(nodes, W1s, W1d, graph_globals, W1u, b1r)

    mesh = plsc.VectorSubcoreMesh(core_axis_name="c", subcore_axis_name="s",
                                  num_cores=_NC, num_subcores=_NS)

    def _gather_entry(ps_hbm, pd_hbm, send_hbm, recv_hbm,
                      out_hbm, idxs, idxr,
                      r0a, r1a, r0b, r1b, sga, swa, sgb, swb):
        _gather_body(ps_hbm, pd_hbm, send_hbm, recv_hbm,
                     out_hbm, idxs, idxr,
                     [(r0a, r1a), (r0b, r1b)],
                     [(sga, swa), (sgb, swb)])

    gather = functools.partial(
        pl.kernel,
        mesh=mesh,
        compiler_params=pltpu.CompilerParams(use_tc_tiling_on_sc=False),
        out_type=jax.ShapeDtypeStruct((_ESL, 2 * _HID), F32),
        scratch_types=[
            pltpu.VMEM((_EPW,), jnp.int32),
            pltpu.VMEM((_EPW,), jnp.int32),
            pltpu.VMEM((_CHUNK, _HID), F32),
            pltpu.VMEM((_CHUNK, _HID), F32),
            pltpu.VMEM((_CHUNK, _HID), F32),
            pltpu.VMEM((_CHUNK, _HID), F32),
            pltpu.SemaphoreType.DMA,
            pltpu.SemaphoreType.DMA,
            pltpu.SemaphoreType.DMA,
            pltpu.SemaphoreType.DMA,
        ],
    )(_gather_entry)
    s2s = [gather(ps, pd, send[k * _ESL:(k + 1) * _ESL],
                  recv[k * _ESL:(k + 1) * _ESL]) for k in range(_NSLICE)]

    parts = []
    for k in range(_NSLICE):
        part = pl.pallas_call(
            _mlp_body,
            grid=(_NBLK,),
            in_specs=[
                pl.BlockSpec((_BE, 2 * _HID), lambda i: (i, 0)),
                pl.BlockSpec((_EDGE_DIM, _BE),
                             lambda i, k=k: (0, i + k * _NBLK)),
                pl.BlockSpec((1, 1, _BE),
                             lambda i, k=k: (i + k * _NBLK, 0, 0)),
                pl.BlockSpec((_N_GRAPHS, _HID), lambda i: (0, 0)),
                pl.BlockSpec((2 * _HID, _HID), lambda i: (0, 0)),
                pl.BlockSpec((_EDGE_DIM, _HID), lambda i: (0, 0)),
                pl.BlockSpec((_HID, _EDGE_DIM), lambda i: (0, 0)),
                pl.BlockSpec((_EDGE_DIM, 1), lambda i: (0, 0)),
            ],
            out_specs=pl.BlockSpec((_EDGE_DIM, _BE), lambda i: (0, i)),
            out_shape=jax.ShapeDtypeStruct((_EDGE_DIM, _ESL), F32),
        )(s2s[k], edges_t, batch3, g, jmat, W1e, W2, b2c)
        parts.append(part)
    out_t = jnp.concatenate(parts, axis=1)
    return out_t.T


# speedup vs baseline: 2.0409x; 1.0037x over previous
import functools

import jax
import jax.numpy as jnp
from jax import lax
from jax.experimental import pallas as pl
from jax.experimental.pallas import tpu as pltpu
from jax.experimental.pallas import tpu_sc as plsc

F32 = jnp.float32
BF16 = jnp.bfloat16
_PREC = lax.Precision.DEFAULT

_N_NODES = 10000
_N_EDGES = 320000
_EDGE_DIM = 16
_NODE_DIM = 128
_HID = 64
_N_GRAPHS = 16

_NSLICE = 5
_ESL = _N_EDGES // _NSLICE

_NC = 2
_NS = 16
_NW = _NC * _NS
_EPW = _ESL // _NW
_CHUNK = 80
_NCHUNK = _EPW // _CHUNK

_BE = 12800
_NBLK = _ESL // _BE
_NBLK_ALL = _N_EDGES // _BE


def _prep_body(nodes_ref, w1s_ref, w1d_ref, gg_ref, w1u_ref, b1_ref,
               ps_ref, pd_ref, g_ref):
    n = nodes_ref[...]
    ps_ref[...] = jnp.dot(n, w1s_ref[...], precision=_PREC,
                          preferred_element_type=F32)
    pd_ref[...] = jnp.dot(n, w1d_ref[...], precision=_PREC,
                          preferred_element_type=F32)
    g_ref[...] = jnp.dot(gg_ref[...], w1u_ref[...], precision=_PREC,
                         preferred_element_type=F32) + b1_ref[...]


def _gather_body(ps_hbm, pd_hbm, send_hbm, recv_hbm,
                 out_hbm, idxs, idxr, bufs, sems):
    wid = lax.axis_index("s") * _NC + lax.axis_index("c")
    base = wid * _EPW
    pltpu.sync_copy(send_hbm.at[pl.ds(base, _EPW)], idxs)
    pltpu.sync_copy(recv_hbm.at[pl.ds(base, _EPW)], idxr)

    def _gather_copies(b, c):
        r0, r1 = bufs[b]
        semg, _ = sems[b]
        sl = pl.ds(c * _CHUNK, _CHUNK)
        return (
            pltpu.make_async_copy(ps_hbm.at[idxs.at[sl]], r0, semg),
            pltpu.make_async_copy(pd_hbm.at[idxr.at[sl]], r1, semg),
        )

    def _write_copies(b, c):
        r0, r1 = bufs[b]
        _, semw = sems[b]
        off = base + c * _CHUNK
        return (
            pltpu.make_async_copy(
                r0, out_hbm.at[pl.ds(off, _CHUNK), pl.ds(0, _HID)], semw),
            pltpu.make_async_copy(
                r1, out_hbm.at[pl.ds(off, _CHUNK), pl.ds(_HID, _HID)], semw),
        )

    def fire(b, c):
        for cp in _gather_copies(b, c):
            cp.start()

    def waitg(b, c):
        for cp in _gather_copies(b, c):
            cp.wait()

    def firew(b, c):
        for cp in _write_copies(b, c):
            cp.start()

    def waitw(b, c):
        for cp in _write_copies(b, c):
            cp.wait()

    fire(0, 0)
    fire(1, 1)
    waitg(0, 0)
    firew(0, 0)
    waitg(1, 1)
    firew(1, 1)
    waitw(0, 0)
    fire(0, 2)

    @pl.loop(1, _NCHUNK // 2)
    def _pair(k):
        c0 = 2 * k
        waitw(1, c0 - 1)
        fire(1, c0 + 1)
        waitg(0, c0)
        firew(0, c0)
        waitg(1, c0 + 1)
        firew(1, c0 + 1)
        waitw(0, c0)
        fire(0, c0 + 2)

    last = _NCHUNK - 1
    waitg(0, last)
    firew(0, last)
    waitw(0, last)
    waitw(1, last - 1)


def _mlp_body(s2_ref, et_ref, b3_ref, g_ref, j_ref, w1e_ref, w2_ref, b2_ref,
              ot_ref):
    brow = jnp.broadcast_to(b3_ref[0], (_N_GRAPHS, _BE))
    onehot_t = (brow == lax.broadcasted_iota(jnp.int32, (_N_GRAPHS, _BE), 0)
                ).astype(F32)
    x = (jnp.dot(s2_ref[...], j_ref[...], precision=_PREC,
                 preferred_element_type=F32)
         + lax.dot_general(et_ref[...], w1e_ref[...],
                           (((0,), (0,)), ((), ())), precision=_PREC,
                           preferred_element_type=F32)
         + lax.dot_general(onehot_t, g_ref[...],
                           (((0,), (0,)), ((), ())), precision=_PREC,
                           preferred_element_type=F32))
    h = jnp.maximum(x, 0.0)
    ot_ref[...] = lax.dot_general(w2_ref[...], h,
                                  (((0,), (1,)), ((), ())), precision=_PREC,
                                  preferred_element_type=F32) + b2_ref[...]


def kernel(nodes, edges, graph_globals, W1, b1, W2, b2, edge_index,
           batch_edges):
    W1e = W1[:_EDGE_DIM]
    W1s = W1[_EDGE_DIM:_EDGE_DIM + _NODE_DIM]
    W1d = W1[_EDGE_DIM + _NODE_DIM:_EDGE_DIM + 2 * _NODE_DIM]
    W1u = W1[_EDGE_DIM + 2 * _NODE_DIM:]
    send = edge_index[0]
    recv = edge_index[1]
    b1r = b1.reshape(1, _HID)
    b2c = b2.reshape(_EDGE_DIM, 1)
    jmat = jnp.concatenate([jnp.eye(_HID, dtype=F32),
                            jnp.eye(_HID, dtype=F32)], axis=0)
    edges_t = edges.T
    batch3 = batch_edges.reshape(_NBLK_ALL, 1, _BE)

    ps, pd, g = pl.pallas_call(
        _prep_body,
        out_shape=[
            jax.ShapeDtypeStruct((_N_NODES, _HID), F32),
            jax.ShapeDtypeStruct((_N_NODES, _HID), F32),
            jax.ShapeDtypeStruct((_N_GRAPHS, _HID), F32),
        ],
    )(nodes, W1s, W1d, graph_globals, W1u, b1r)

    mesh = plsc.VectorSubcoreMesh(core_axis_name="c", subcore_axis_name="s",
                                  num_cores=_NC, num_subcores=_NS)

    def _gather_entry(ps_hbm, pd_hbm, send_hbm, recv_hbm,
                      out_hbm, idxs, idxr,
                      r0a, r1a, r0b, r1b, sga, swa, sgb, swb):
        _gather_body(ps_hbm, pd_hbm, send_hbm, recv_hbm,
                     out_hbm, idxs, idxr,
                     [(r0a, r1a), (r0b, r1b)],
                     [(sga, swa), (sgb, swb)])

    gather = functools.partial(
        pl.kernel,
        mesh=mesh,
        compiler_params=pltpu.CompilerParams(use_tc_tiling_on_sc=False),
        out_type=jax.ShapeDtypeStruct((_ESL, 2 * _HID), F32),
        scratch_types=[
            pltpu.VMEM((_EPW,), jnp.int32),
            pltpu.VMEM((_EPW,), jnp.int32),
            pltpu.VMEM((_CHUNK, _HID), F32),
            pltpu.VMEM((_CHUNK, _HID), F32),
            pltpu.VMEM((_CHUNK, _HID), F32),
            pltpu.VMEM((_CHUNK, _HID), F32),
            pltpu.SemaphoreType.DMA,
            pltpu.SemaphoreType.DMA,
            pltpu.SemaphoreType.DMA,
            pltpu.SemaphoreType.DMA,
        ],
    )(_gather_entry)
    s2s = [gather(ps, pd, send[k * _ESL:(k + 1) * _ESL],
                  recv[k * _ESL:(k + 1) * _ESL]) for k in range(_NSLICE)]

    parts = []
    for k in range(_NSLICE):
        part = pl.pallas_call(
            _mlp_body,
            grid=(_NBLK,),
            in_specs=[
                pl.BlockSpec((_BE, 2 * _HID), lambda i: (i, 0)),
                pl.BlockSpec((_EDGE_DIM, _BE),
                             lambda i, k=k: (0, i + k * _NBLK)),
                pl.BlockSpec((1, 1, _BE),
                             lambda i, k=k: (i + k * _NBLK, 0, 0)),
                pl.BlockSpec((_N_GRAPHS, _HID), lambda i: (0, 0)),
                pl.BlockSpec((2 * _HID, _HID), lambda i: (0, 0)),
                pl.BlockSpec((_EDGE_DIM, _HID), lambda i: (0, 0)),
                pl.BlockSpec((_HID, _EDGE_DIM), lambda i: (0, 0)),
                pl.BlockSpec((_EDGE_DIM, 1), lambda i: (0, 0)),
            ],
            out_specs=pl.BlockSpec((_EDGE_DIM, _BE), lambda i: (0, i)),
            out_shape=jax.ShapeDtypeStruct((_EDGE_DIM, _ESL), F32),
        )(s2s[k], edges_t, batch3, g, jmat, W1e, W2, b2c)
        parts.append(part)
    out_t = jnp.concatenate(parts, axis=1)
    return out_t.T


# speedup vs baseline: 2.0446x; 1.0018x over previous
import functools

import jax
import jax.numpy as jnp
from jax import lax
from jax.experimental import pallas as pl
from jax.experimental.pallas import tpu as pltpu
from jax.experimental.pallas import tpu_sc as plsc

F32 = jnp.float32
BF16 = jnp.bfloat16
_PREC = lax.Precision.DEFAULT

_N_NODES = 10000
_N_EDGES = 320000
_EDGE_DIM = 16
_NODE_DIM = 128
_HID = 64
_N_GRAPHS = 16

_NSLICE = 5
_ESL = _N_EDGES // _NSLICE

_NC = 2
_NS = 16
_NW = _NC * _NS
_EPW = _ESL // _NW
_CHUNK = 80
_NCHUNK = _EPW // _CHUNK

_BE = 12800
_NBLK = _ESL // _BE
_NBLK_ALL = _N_EDGES // _BE


def _prep_body(nodes_ref, w1s_ref, w1d_ref, gg_ref, w1u_ref, b1_ref,
               ps_ref, pd_ref, g_ref):
    n = nodes_ref[...]
    ps_ref[...] = jnp.dot(n, w1s_ref[...], precision=_PREC,
                          preferred_element_type=F32)
    pd_ref[...] = jnp.dot(n, w1d_ref[...], precision=_PREC,
                          preferred_element_type=F32)
    g_ref[...] = jnp.dot(gg_ref[...], w1u_ref[...], precision=_PREC,
                         preferred_element_type=F32) + b1_ref[...]


def _gather_body(ps_hbm, pd_hbm, send_hbm, recv_hbm,
                 out_hbm, idxs, idxr, bufs, sems):
    wid = lax.axis_index("s") * _NC + lax.axis_index("c")
    base = wid * _EPW
    pltpu.sync_copy(send_hbm.at[pl.ds(base, _EPW)], idxs)
    pltpu.sync_copy(recv_hbm.at[pl.ds(base, _EPW)], idxr)

    def _gather_copies(b, c):
        r0, r1 = bufs[b]
        semg, _ = sems[b]
        sl = pl.ds(c * _CHUNK, _CHUNK)
        return (
            pltpu.make_async_copy(ps_hbm.at[idxs.at[sl]], r0, semg),
            pltpu.make_async_copy(pd_hbm.at[idxr.at[sl]], r1, semg),
        )

    def _write_copies(b, c):
        r0, r1 = bufs[b]
        _, semw = sems[b]
        off = base + c * _CHUNK
        return (
            pltpu.make_async_copy(
                r0, out_hbm.at[pl.ds(off, _CHUNK), pl.ds(0, _HID)], semw),
            pltpu.make_async_copy(
                r1, out_hbm.at[pl.ds(off, _CHUNK), pl.ds(_HID, _HID)], semw),
        )

    def fire(b, c):
        for cp in _gather_copies(b, c):
            cp.start()

    def waitg(b, c):
        for cp in _gather_copies(b, c):
            cp.wait()

    def firew(b, c):
        for cp in _write_copies(b, c):
            cp.start()

    def waitw(b, c):
        for cp in _write_copies(b, c):
            cp.wait()

    fire(0, 0)
    fire(1, 1)
    waitg(0, 0)
    firew(0, 0)
    waitg(1, 1)
    firew(1, 1)
    waitw(0, 0)
    fire(0, 2)

    @pl.loop(1, _NCHUNK // 2)
    def _pair(k):
        c0 = 2 * k
        waitw(1, c0 - 1)
        fire(1, c0 + 1)
        waitg(0, c0)
        firew(0, c0)
        waitg(1, c0 + 1)
        firew(1, c0 + 1)
        waitw(0, c0)
        fire(0, c0 + 2)

    last = _NCHUNK - 1
    waitg(0, last)
    firew(0, last)
    waitw(0, last)
    waitw(1, last - 1)


def _mlp_body(s2_ref, et_ref, b3_ref, g_ref, j_ref, w1e_ref, w2_ref, b2_ref,
              ot_ref):
    brow = jnp.broadcast_to(b3_ref[0], (_N_GRAPHS, _BE))
    onehot_t = (brow == lax.broadcasted_iota(jnp.int32, (_N_GRAPHS, _BE), 0)
                ).astype(BF16)
    x = (jnp.dot(s2_ref[...].astype(BF16), j_ref[...],
                 preferred_element_type=F32)
         + lax.dot_general(et_ref[...].astype(BF16), w1e_ref[...],
                           (((0,), (0,)), ((), ())),
                           preferred_element_type=F32)
         + lax.dot_general(onehot_t, g_ref[...],
                           (((0,), (0,)), ((), ())),
                           preferred_element_type=F32))
    h = jnp.maximum(x, 0.0)
    ot_ref[...] = lax.dot_general(w2_ref[...], h,
                                  (((0,), (1,)), ((), ())), precision=_PREC,
                                  preferred_element_type=F32) + b2_ref[...]


def kernel(nodes, edges, graph_globals, W1, b1, W2, b2, edge_index,
           batch_edges):
    W1e = W1[:_EDGE_DIM]
    W1s = W1[_EDGE_DIM:_EDGE_DIM + _NODE_DIM]
    W1d = W1[_EDGE_DIM + _NODE_DIM:_EDGE_DIM + 2 * _NODE_DIM]
    W1u = W1[_EDGE_DIM + 2 * _NODE_DIM:]
    send = edge_index[0]
    recv = edge_index[1]
    b1r = b1.reshape(1, _HID)
    b2c = b2.reshape(_EDGE_DIM, 1)
    jmat = jnp.concatenate([jnp.eye(_HID, dtype=BF16),
                            jnp.eye(_HID, dtype=BF16)], axis=0)
    w1e_bf = W1e.astype(BF16)
    edges_t = edges.T
    batch3 = batch_edges.reshape(_NBLK_ALL, 1, _BE)

    ps, pd, g = pl.pallas_call(
        _prep_body,
        out_shape=[
            jax.ShapeDtypeStruct((_N_NODES, _HID), F32),
            jax.ShapeDtypeStruct((_N_NODES, _HID), F32),
            jax.ShapeDtypeStruct((_N_GRAPHS, _HID), F32),
        ],
    )(nodes, W1s, W1d, graph_globals, W1u, b1r)

    mesh = plsc.VectorSubcoreMesh(core_axis_name="c", subcore_axis_name="s",
                                  num_cores=_NC, num_subcores=_NS)

    def _gather_entry(ps_hbm, pd_hbm, send_hbm, recv_hbm,
                      out_hbm, idxs, idxr,
                      r0a, r1a, r0b, r1b, sga, swa, sgb, swb):
        _gather_body(ps_hbm, pd_hbm, send_hbm, recv_hbm,
                     out_hbm, idxs, idxr,
                     [(r0a, r1a), (r0b, r1b)],
                     [(sga, swa), (sgb, swb)])

    gather = functools.partial(
        pl.kernel,
        mesh=mesh,
        compiler_params=pltpu.CompilerParams(use_tc_tiling_on_sc=False),
        out_type=jax.ShapeDtypeStruct((_ESL, 2 * _HID), F32),
        scratch_types=[
            pltpu.VMEM((_EPW,), jnp.int32),
            pltpu.VMEM((_EPW,), jnp.int32),
            pltpu.VMEM((_CHUNK, _HID), F32),
            pltpu.VMEM((_CHUNK, _HID), F32),
            pltpu.VMEM((_CHUNK, _HID), F32),
            pltpu.VMEM((_CHUNK, _HID), F32),
            pltpu.SemaphoreType.DMA,
            pltpu.SemaphoreType.DMA,
            pltpu.SemaphoreType.DMA,
            pltpu.SemaphoreType.DMA,
        ],
    )(_gather_entry)
    s2s = [gather(ps, pd, send[k * _ESL:(k + 1) * _ESL],
                  recv[k * _ESL:(k + 1) * _ESL]) for k in range(_NSLICE)]

    parts = []
    for k in range(_NSLICE):
        part = pl.pallas_call(
            _mlp_body,
            grid=(_NBLK,),
            in_specs=[
                pl.BlockSpec((_BE, 2 * _HID), lambda i: (i, 0)),
                pl.BlockSpec((_EDGE_DIM, _BE),
                             lambda i, k=k: (0, i + k * _NBLK)),
                pl.BlockSpec((1, 1, _BE),
                             lambda i, k=k: (i + k * _NBLK, 0, 0)),
                pl.BlockSpec((_N_GRAPHS, _HID), lambda i: (0, 0)),
                pl.BlockSpec((2 * _HID, _HID), lambda i: (0, 0)),
                pl.BlockSpec((_EDGE_DIM, _HID), lambda i: (0, 0)),
                pl.BlockSpec((_HID, _EDGE_DIM), lambda i: (0, 0)),
                pl.BlockSpec((_EDGE_DIM, 1), lambda i: (0, 0)),
            ],
            out_specs=pl.BlockSpec((_EDGE_DIM, _BE), lambda i: (0, i)),
            out_shape=jax.ShapeDtypeStruct((_EDGE_DIM, _ESL), F32),
        )(s2s[k], edges_t, batch3, g.astype(BF16), jmat, w1e_bf, W2, b2c)
        parts.append(part)
    out_t = jnp.concatenate(parts, axis=1)
    return out_t.T
